# baseline (speedup 1.0000x reference)
import functools

import jax
import jax.numpy as jnp
from jax import lax
from jax.experimental import pallas as pl
from jax.experimental.pallas import tpu as pltpu
from jax.experimental.pallas import tpu_sc as plsc

VOCAB = 1000
EMBED_DIM = 32
BATCH = 16384
PAD_DIM = 128
CHUNK = 128
ROWS_PER_LINE = PAD_DIM // EMBED_DIM
LANES = 16


@functools.lru_cache(maxsize=None)
def _build():
    info = plsc.get_sparse_core_info()
    nc, ns = info.num_cores, info.num_subcores
    nw = nc * ns
    b_per_w = BATCH // nw
    n_chunks = b_per_w // CHUNK
    lines_per_chunk = CHUNK // ROWS_PER_LINE

    mesh = plsc.VectorSubcoreMesh(core_axis_name="c", subcore_axis_name="s")

    @functools.partial(
        pl.kernel,
        mesh=mesh,
        out_type=jax.ShapeDtypeStruct((BATCH // ROWS_PER_LINE, PAD_DIM),
                                      jnp.float32),
        scratch_types=[
            pltpu.VMEM((b_per_w,), jnp.int32),
            pltpu.VMEM((2, CHUNK, PAD_DIM), jnp.float32),
            pltpu.VMEM((2, lines_per_chunk, PAD_DIM), jnp.float32),
            pltpu.VMEM_SHARED((VOCAB, PAD_DIM), jnp.float32),
            pltpu.SemaphoreType.DMA,
            pltpu.SemaphoreType.DMA,
            pltpu.SemaphoreType.DMA,
        ],
    )
    def gather_kernel(idx_hbm, table_hbm, out_hbm, idx_v, bufs, packed,
                      table_sp, gsem, wsem0, wsem1):
        sid = lax.axis_index("s")
        wid = sid * nc + lax.axis_index("c")
        base = wid * b_per_w
        lbase = wid * (b_per_w // ROWS_PER_LINE)
        @pl.when(sid == 0)
        def _():
            pltpu.sync_copy(table_hbm, table_sp)
        pltpu.sync_copy(idx_hbm.at[pl.ds(base, b_per_w)], idx_v)
        plsc.subcore_barrier()

        wsems = (wsem0, wsem1)
        gathers = [None, None]
        writes = [None, None]
        gathers[0] = pltpu.async_copy(
            table_sp.at[idx_v.at[pl.ds(0, CHUNK)]], bufs.at[0], gsem)
        for k in range(n_chunks):
            b = k % 2
            gathers[b].wait()
            if k + 1 < n_chunks:
                nb = (k + 1) % 2
                gathers[nb] = pltpu.async_copy(
                    table_sp.at[idx_v.at[pl.ds((k + 1) * CHUNK, CHUNK)]],
                    bufs.at[nb], gsem)
            if writes[b] is not None:
                writes[b].wait()

            def compact(line, carry, b=b):
                for q in range(ROWS_PER_LINE):
                    row = line * ROWS_PER_LINE + q
                    for h in range(EMBED_DIM // LANES):
                        packed[b, line,
                               pl.ds(q * EMBED_DIM + h * LANES, LANES)] = (
                            bufs[b, row, pl.ds(h * LANES, LANES)])
                return carry

            lax.fori_loop(0, lines_per_chunk, compact, 0)
            writes[b] = pltpu.async_copy(
                packed.at[b],
                out_hbm.at[pl.ds(lbase + k * lines_per_chunk,
                                 lines_per_chunk)],
                wsems[b])
        for w in writes:
            if w is not None:
                w.wait()

    return gather_kernel


def kernel(user_id, table):
    table_padded = jnp.pad(table, ((0, 0), (0, PAD_DIM - EMBED_DIM)))
    out_packed = _build()(user_id, table_padded)
    return out_packed.reshape(BATCH, EMBED_DIM)


# speedup vs baseline: 1.2186x; 1.2186x over previous
import functools

import jax
import jax.numpy as jnp
from jax import lax
from jax.experimental import pallas as pl
from jax.experimental.pallas import tpu as pltpu
from jax.experimental.pallas import tpu_sc as plsc

VOCAB = 1000
VOCAB_PAD = 1024
EMBED_DIM = 32
BATCH = 16384
PAD_DIM = 128
CHUNK = 64


@functools.lru_cache(maxsize=None)
def _build():
    info = plsc.get_sparse_core_info()
    nc, ns = info.num_cores, info.num_subcores
    nw = nc * ns
    b_per_w = BATCH // nw

    mesh = plsc.VectorSubcoreMesh(core_axis_name="c", subcore_axis_name="s")

    @functools.partial(
        pl.kernel,
        mesh=mesh,
        out_type=jax.ShapeDtypeStruct((BATCH, PAD_DIM), jnp.float32),
        scratch_types=[
            pltpu.VMEM((b_per_w,), jnp.int32),
            pltpu.VMEM((2, CHUNK, PAD_DIM), jnp.float32),
            pltpu.VMEM_SHARED((VOCAB_PAD, PAD_DIM), jnp.float32),
            pltpu.SemaphoreType.DMA,
            pltpu.SemaphoreType.DMA,
            pltpu.SemaphoreType.DMA,
        ],
    )
    def gather_kernel(idx_hbm, table_hbm, out_hbm, idx_v, rows_v, table_sp,
                      gsem, wsem0, wsem1):
        sid = lax.axis_index("s")
        wid = sid * nc + lax.axis_index("c")
        base = wid * b_per_w
        n_chunks = b_per_w // CHUNK
        @pl.when(sid < 8)
        def _():
            pltpu.sync_copy(
                table_hbm.at[pl.ds(sid * (VOCAB_PAD // 8), VOCAB_PAD // 8)],
                table_sp.at[pl.ds(sid * (VOCAB_PAD // 8), VOCAB_PAD // 8)])
        pltpu.sync_copy(idx_hbm.at[pl.ds(base, b_per_w)], idx_v)
        plsc.subcore_barrier()
        wsems = (wsem0, wsem1)
        writes = [None, None]
        for k in range(n_chunks):
            b = k % 2
            if writes[b] is not None:
                writes[b].wait()
            pltpu.async_copy(
                table_sp.at[idx_v.at[pl.ds(k * CHUNK, CHUNK)]],
                rows_v.at[b], gsem).wait()
            writes[b] = pltpu.async_copy(
                rows_v.at[b], out_hbm.at[pl.ds(base + k * CHUNK, CHUNK)],
                wsems[b])
        for w in writes:
            if w is not None:
                w.wait()

    return gather_kernel


def kernel(user_id, table):
    table_padded = jnp.pad(
        table, ((0, VOCAB_PAD - VOCAB), (0, PAD_DIM - EMBED_DIM)))
    out_padded = _build()(user_id, table_padded)
    return out_padded[:, :EMBED_DIM]
